# baseline (speedup 1.0000x reference)
import jax
import jax.numpy as jnp
from jax import lax
from jax.experimental import pallas as pl
from jax.experimental.pallas import tpu as pltpu
from jax.experimental.pallas import tpu_sc as plsc

IGNORE = 19
CE_W = 0.5
LV_W = 0.5
NCLS = 10
C = 20
NB = 2048
HSIZE = 2 * NCLS * NB

BATCH, HDIM, WDIM = 4, 512, 512
NPIX = BATCH * HDIM * WDIM
HBLK = 32
GRID_H = HDIM // HBLK
ROWS_STEP = HBLK * WDIM // 128

NIDX = NCLS * NPIX
NSC, NTEC = 2, 16
NW = NSC * NTEC
PER_W = NIDX // NW
CHUNK = 32768
NCHUNK = PER_W // CHUNK
GROUPS = CHUNK // 2048


def _p1_body(x_ref, t_ref, idx_ref, ce_ref):
    first = jnp.logical_and(pl.program_id(0) == 0, pl.program_id(1) == 0)

    @pl.when(first)
    def _():
        ce_ref[...] = jnp.zeros_like(ce_ref)

    x = x_ref[0]
    t = t_ref[0]
    m = jnp.max(x, axis=0)
    ex = jnp.exp(x - m[None])
    s = jnp.sum(ex, axis=0)
    lse = jnp.log(s) + m
    xt = jnp.zeros((HBLK, WDIM), jnp.float32)
    fgf = []
    for c in range(C):
        cmp = t == c
        xt = xt + jnp.where(cmp, x[c], 0.0)
        if c < NCLS:
            fgf.append(cmp.astype(jnp.float32))
    maskf = (t != IGNORE).astype(jnp.float32)
    nll = (lse - xt) * maskf
    ce_ref[0] += jnp.sum(nll.reshape(HBLK, WDIM // 128, 128), axis=1)
    ce_ref[1] += jnp.sum(maskf.reshape(HBLK, WDIM // 128, 128), axis=1)

    s_inv = 1.0 / s
    for c in range(NCLS):
        p = ex[c] * s_inv
        e = jnp.abs(fgf[c] - p) * maskf
        idx_f = jnp.minimum(e * NB, NB - 1.0) + fgf[c] * (NCLS * NB) + c * NB
        idx_ref[c] = idx_f.astype(jnp.int32).reshape(ROWS_STEP, 128)


def _p3_body(hist_ref, ce_ref, out_ref):
    tot = hist_ref[pl.ds(0, C)]
    for w in range(1, NW):
        tot += hist_ref[pl.ds(w * C, C)]
    fgc = tot[NCLS:]
    cnt = tot[:NCLS] + fgc

    def cum(v):
        sh = 1
        while sh < NB:
            z = jnp.zeros((NCLS, sh), jnp.float32)
            v = v + jnp.concatenate([z, v[:, :-sh]], axis=1)
            sh *= 2
        return v

    cum_c = cum(cnt)
    cum_f = cum(fgc)
    tot_c = cum_c[:, -1:]
    tot_f = cum_f[:, -1:]
    n_b = tot_c - cum_c + cnt
    f_b = tot_f - cum_f + fgc
    gts = tot_f
    j_end = 1.0 - (gts - f_b) / jnp.maximum(gts + n_b - f_b, 1.0)
    j_sta = 1.0 - (gts - (f_b - fgc)) / jnp.maximum(
        gts + (n_b - cnt) - (f_b - fgc), 1.0)
    eb = (lax.broadcasted_iota(jnp.int32, (NCLS, NB), 1).astype(jnp.float32)
          + 0.5) / NB
    term = jnp.sum(eb * (j_end - j_sta), axis=1, keepdims=True)
    lv = jnp.sum(jnp.where(gts > 0, term, 0.0)) / NCLS
    ce = jnp.sum(ce_ref[0]) / jnp.sum(ce_ref[1])
    out_ref[...] = jnp.full((8, 128), CE_W * ce + LV_W * lv, jnp.float32)


def _sc_hist(idx_hbm, zeros_hbm, out_hbm, buf0, buf1, hist_v, sem0, sem1):
    cid = lax.axis_index("c")
    sid = lax.axis_index("s")
    wid = sid * NSC + cid
    base = wid * PER_W

    pltpu.sync_copy(zeros_hbm, hist_v)

    bufs = (buf0, buf1)
    sems = (sem0, sem1)

    def load(ch, slot):
        return pltpu.make_async_copy(
            idx_hbm.at[pl.ds(base + ch * CHUNK, CHUNK)], bufs[slot], sems[slot])

    ones = jnp.ones((16,), jnp.float32)

    def scatter_chunk(buf):
        @plsc.parallel_loop(0, CHUNK // 16, unroll=8)
        def _(t):
            vec = buf[pl.ds(t * 16, 16)]
            plsc.addupdate_scatter(hist_v, [vec], ones)

    load(0, 0).start()
    for ch in range(NCHUNK):
        slot = ch % 2
        if ch + 1 < NCHUNK:
            load(ch + 1, 1 - slot).start()
        load(ch, slot).wait()
        scatter_chunk(bufs[slot])

    pltpu.sync_copy(hist_v, out_hbm.at[wid])


def kernel(inputs, targets):
    targets = targets.astype(jnp.int32)

    idx, ce_parts = pl.pallas_call(
        _p1_body,
        grid=(BATCH, GRID_H),
        in_specs=[
            pl.BlockSpec((1, C, HBLK, WDIM), lambda b, h: (b, 0, h, 0)),
            pl.BlockSpec((1, HBLK, WDIM), lambda b, h: (b, h, 0)),
        ],
        out_specs=[
            pl.BlockSpec((NCLS, ROWS_STEP, 128),
                         lambda b, h: (0, b * GRID_H + h, 0)),
            pl.BlockSpec((2, HBLK, 128), lambda b, h: (0, 0, 0)),
        ],
        out_shape=[
            jax.ShapeDtypeStruct((NCLS, NPIX // 128, 128), jnp.int32),
            jax.ShapeDtypeStruct((2, HBLK, 128), jnp.float32),
        ],
    )(inputs, targets)

    idx1d = idx.reshape(NIDX)
    zeros = jnp.zeros((HSIZE,), jnp.float32)

    sc_fn = pl.kernel(
        _sc_hist,
        out_type=jax.ShapeDtypeStruct((NW, HSIZE), jnp.float32),
        mesh=plsc.VectorSubcoreMesh(core_axis_name="c", subcore_axis_name="s"),
        compiler_params=pltpu.CompilerParams(needs_layout_passes=False),
        scratch_types=[
            pltpu.VMEM((CHUNK,), jnp.int32),
            pltpu.VMEM((CHUNK,), jnp.int32),
            pltpu.VMEM((HSIZE,), jnp.float32),
            pltpu.SemaphoreType.DMA,
            pltpu.SemaphoreType.DMA,
        ],
    )
    hist = sc_fn(idx1d, zeros)

    out = pl.pallas_call(
        _p3_body,
        in_specs=[
            pl.BlockSpec((NW * C, NB), lambda: (0, 0)),
            pl.BlockSpec((2, HBLK, 128), lambda: (0, 0, 0)),
        ],
        out_specs=pl.BlockSpec((8, 128), lambda: (0, 0)),
        out_shape=jax.ShapeDtypeStruct((8, 128), jnp.float32),
    )(hist.reshape(NW * C, NB), ce_parts)

    return out[0, 0]


# speedup vs baseline: 1.1135x; 1.1135x over previous
import jax
import jax.numpy as jnp
from jax import lax
from jax.experimental import pallas as pl
from jax.experimental.pallas import tpu as pltpu
from jax.experimental.pallas import tpu_sc as plsc

IGNORE = 19
CE_W = 0.5
LV_W = 0.5
NCLS = 10
C = 20
NB = 2048
HSIZE = 2 * NCLS * NB

BATCH, HDIM, WDIM = 4, 512, 512
NPIX = BATCH * HDIM * WDIM
HBLK = 64
GRID_H = HDIM // HBLK
ROWS_STEP = HBLK * WDIM // 128

NIDX = NCLS * NPIX
NSC, NTEC = 2, 16
NW = NSC * NTEC
PER_W = NIDX // NW
CHUNK = 32768
NCHUNK = PER_W // CHUNK
GROUPS = CHUNK // 2048


def _p1_body(x_ref, t_ref, idx_ref, ce_ref):
    first = jnp.logical_and(pl.program_id(0) == 0, pl.program_id(1) == 0)

    @pl.when(first)
    def _():
        ce_ref[...] = jnp.zeros_like(ce_ref)

    x = x_ref[0]
    t = t_ref[0]
    m = jnp.max(x, axis=0)
    ex = jnp.exp(x - m[None])
    s = jnp.sum(ex, axis=0)
    lse = jnp.log(s) + m
    xt = jnp.zeros((HBLK, WDIM), jnp.float32)
    fgf = []
    for c in range(C):
        cmp = t == c
        xt = xt + jnp.where(cmp, x[c], 0.0)
        if c < NCLS:
            fgf.append(cmp.astype(jnp.float32))
    maskf = (t != IGNORE).astype(jnp.float32)
    nll = (lse - xt) * maskf
    ce_ref[0] += jnp.sum(nll.reshape(HBLK, WDIM // 128, 128), axis=1)
    ce_ref[1] += jnp.sum(maskf.reshape(HBLK, WDIM // 128, 128), axis=1)

    s_inv = 1.0 / s
    for c in range(NCLS):
        p = ex[c] * s_inv
        e = jnp.abs(fgf[c] - p) * maskf
        idx_f = jnp.minimum(e * NB, NB - 1.0) + fgf[c] * (NCLS * NB) + c * NB
        idx = idx_f.astype(jnp.int32)
        half = ROWS_STEP // 2
        idx_ref[c, :half] = idx[: HBLK // 2].reshape(half, 128)
        idx_ref[c, half:] = idx[HBLK // 2:].reshape(half, 128)


def _p3_body(hist_ref, ce_ref, out_ref):
    tot = hist_ref[pl.ds(0, C)]
    for w in range(1, NW):
        tot += hist_ref[pl.ds(w * C, C)]
    fgc = tot[NCLS:]
    cnt = tot[:NCLS] + fgc

    def cum(v):
        sh = 1
        while sh < NB:
            z = jnp.zeros((NCLS, sh), jnp.float32)
            v = v + jnp.concatenate([z, v[:, :-sh]], axis=1)
            sh *= 2
        return v

    cum_c = cum(cnt)
    cum_f = cum(fgc)
    tot_c = cum_c[:, -1:]
    tot_f = cum_f[:, -1:]
    n_b = tot_c - cum_c + cnt
    f_b = tot_f - cum_f + fgc
    gts = tot_f
    j_end = 1.0 - (gts - f_b) / jnp.maximum(gts + n_b - f_b, 1.0)
    j_sta = 1.0 - (gts - (f_b - fgc)) / jnp.maximum(
        gts + (n_b - cnt) - (f_b - fgc), 1.0)
    eb = (lax.broadcasted_iota(jnp.int32, (NCLS, NB), 1).astype(jnp.float32)
          + 0.5) / NB
    term = jnp.sum(eb * (j_end - j_sta), axis=1, keepdims=True)
    lv = jnp.sum(jnp.where(gts > 0, term, 0.0)) / NCLS
    ce = jnp.sum(ce_ref[0]) / jnp.sum(ce_ref[1])
    out_ref[...] = jnp.full((8, 128), CE_W * ce + LV_W * lv, jnp.float32)


def _sc_hist(idx_hbm, zeros_hbm, out_hbm, buf0, buf1, hist_v, sem0, sem1):
    cid = lax.axis_index("c")
    sid = lax.axis_index("s")
    wid = sid * NSC + cid
    base = wid * PER_W

    pltpu.sync_copy(zeros_hbm, hist_v)

    bufs = (buf0, buf1)
    sems = (sem0, sem1)

    def load(ch, slot):
        return pltpu.make_async_copy(
            idx_hbm.at[pl.ds(base + ch * CHUNK, CHUNK)], bufs[slot], sems[slot])

    ones = jnp.ones((16,), jnp.float32)

    def scatter_chunk(buf):
        @plsc.parallel_loop(0, CHUNK // 16, unroll=8)
        def _(t):
            vec = buf[pl.ds(t * 16, 16)]
            plsc.addupdate_scatter(hist_v, [vec], ones)

    load(0, 0).start()
    for ch in range(NCHUNK):
        slot = ch % 2
        if ch + 1 < NCHUNK:
            load(ch + 1, 1 - slot).start()
        load(ch, slot).wait()
        scatter_chunk(bufs[slot])

    pltpu.sync_copy(hist_v, out_hbm.at[wid])


def kernel(inputs, targets):
    targets = targets.astype(jnp.int32)

    idx, ce_parts = pl.pallas_call(
        _p1_body,
        grid=(BATCH, GRID_H),
        in_specs=[
            pl.BlockSpec((1, C, HBLK, WDIM), lambda b, h: (b, 0, h, 0)),
            pl.BlockSpec((1, HBLK, WDIM), lambda b, h: (b, h, 0)),
        ],
        out_specs=[
            pl.BlockSpec((NCLS, ROWS_STEP, 128),
                         lambda b, h: (0, b * GRID_H + h, 0)),
            pl.BlockSpec((2, HBLK, 128), lambda b, h: (0, 0, 0)),
        ],
        out_shape=[
            jax.ShapeDtypeStruct((NCLS, NPIX // 128, 128), jnp.int32),
            jax.ShapeDtypeStruct((2, HBLK, 128), jnp.float32),
        ],
    )(inputs, targets)

    idx1d = idx.reshape(NIDX)
    zeros = jnp.zeros((HSIZE,), jnp.float32)

    sc_fn = pl.kernel(
        _sc_hist,
        out_type=jax.ShapeDtypeStruct((NW, HSIZE), jnp.float32),
        mesh=plsc.VectorSubcoreMesh(core_axis_name="c", subcore_axis_name="s"),
        compiler_params=pltpu.CompilerParams(needs_layout_passes=False),
        scratch_types=[
            pltpu.VMEM((CHUNK,), jnp.int32),
            pltpu.VMEM((CHUNK,), jnp.int32),
            pltpu.VMEM((HSIZE,), jnp.float32),
            pltpu.SemaphoreType.DMA,
            pltpu.SemaphoreType.DMA,
        ],
    )
    hist = sc_fn(idx1d, zeros)

    out = pl.pallas_call(
        _p3_body,
        in_specs=[
            pl.BlockSpec((NW * C, NB), lambda: (0, 0)),
            pl.BlockSpec((2, HBLK, 128), lambda: (0, 0, 0)),
        ],
        out_specs=pl.BlockSpec((8, 128), lambda: (0, 0)),
        out_shape=jax.ShapeDtypeStruct((8, 128), jnp.float32),
    )(hist.reshape(NW * C, NB), ce_parts)

    return out[0, 0]
